# baseline (speedup 1.0000x reference)
import functools

import jax
import jax.numpy as jnp
from jax import lax
from jax.experimental import pallas as pl
from jax.experimental.pallas import tpu as pltpu
from jax.experimental.pallas import tpu_sc as plsc

B, L, D = 1024, 512, 64
DP = 128
V = 1000000
VP = V // DP
VT = VP * DP
NC, NS = 2, 16
NW = NC * NS
ROWS_PER_W = B // NW
NQ = 4
QL = L // NQ
LANES = 16
COLS_PER_W = VP // NW


def _transpose_to_pairs(stage, block, npairs):
    lane = jnp.arange(LANES, dtype=jnp.int32)
    rows = [(k % 4) * LANES + lane for k in range(8)]
    zero = jnp.zeros((LANES,), jnp.int32)

    for p in range(npairs):
        cols = [zero + (2 * p), zero + (2 * p + 1)]
        for k in range(8):
            v = plsc.load_gather(stage, [rows[k], cols[k // 4]])
            block[p, pl.ds(k * LANES, LANES)] = v


def _relayout_body(tokt_hbm, pair_hbm, stage0, stage1, stage_t,
                   block0, block1, sem_r, sem_w):
    stages = [stage0, stage1]
    blocks = [block0, block1]
    c = lax.axis_index("c")
    s = lax.axis_index("s")
    wid = s * NC + c
    ncols = COLS_PER_W + jnp.where(wid < VP - COLS_PER_W * NW, 1, 0)
    NP = DP // 2

    def read(i, buf):
        v = wid + NW * i
        return pltpu.async_copy(tokt_hbm.at[:, pl.ds(v * DP, DP)], buf, sem_r)

    def drain(sem, buf):
        pltpu.make_async_copy(tokt_hbm.at[:, pl.ds(0, DP)], buf, sem).wait()

    read(0, stages[0])
    read(1, stages[1])

    def do_col(i, _):
        par = i % 2

        @pl.when(par == 0)
        def _even():
            drain(sem_r, stages[0])

            @pl.when(i >= 2)
            def _():
                drain(sem_w, blocks[0])

            _transpose_to_pairs(stages[0], blocks[0], NP)
            pltpu.async_copy(
                blocks[0], pair_hbm.at[pl.ds((wid + NW * i) * NP, NP)], sem_w
            )

            @pl.when(i + 2 < ncols)
            def _():
                read(i + 2, stages[0])

        @pl.when(par == 1)
        def _odd():
            drain(sem_r, stages[1])

            @pl.when(i >= 2)
            def _():
                drain(sem_w, blocks[1])

            _transpose_to_pairs(stages[1], blocks[1], NP)
            pltpu.async_copy(
                blocks[1], pair_hbm.at[pl.ds((wid + NW * i) * NP, NP)], sem_w
            )

            @pl.when(i + 2 < ncols)
            def _():
                read(i + 2, stages[1])

        return 0

    lax.fori_loop(0, ncols, do_col, 0)
    drain(sem_w, blocks[0])
    drain(sem_w, blocks[1])

    @pl.when(wid == NW - 1)
    def _tail():
        for d in range(D):
            pltpu.sync_copy(tokt_hbm.at[d, pl.ds(VT, V - VT)], stage_t.at[d])
        _transpose_to_pairs(stage_t, blocks[0], (V - VT) // 2)
        pltpu.async_copy(
            blocks[0].at[pl.ds(0, (V - VT) // 2)],
            pair_hbm.at[pl.ds(VT // 2, (V - VT) // 2)],
            sem_w,
        ).wait()


_relayout = functools.partial(
    pl.kernel,
    out_type=jax.ShapeDtypeStruct((V // 2, DP), jnp.float32),
    mesh=plsc.VectorSubcoreMesh(core_axis_name="c", subcore_axis_name="s"),
    scratch_types=[
        pltpu.VMEM((D, DP), jnp.float32),
        pltpu.VMEM((D, DP), jnp.float32),
        pltpu.VMEM((D, D), jnp.float32),
        pltpu.VMEM((DP // 2, DP), jnp.float32),
        pltpu.VMEM((DP // 2, DP), jnp.float32),
        pltpu.SemaphoreType.DMA,
        pltpu.SemaphoreType.DMA,
    ],
    compiler_params=pltpu.CompilerParams(
        use_tc_tiling_on_sc=True, needs_layout_passes=False
    ),
)(_relayout_body)


def _gather_body(x_hbm, pair_hbm, pos_hbm, out_hbm,
                 idx_v, idx2_v, bufa, bufb, pos_v, sem_g, sem_w):
    c = lax.axis_index("c")
    s = lax.axis_index("s")
    wid = s * NC + c
    base = wid * ROWS_PER_W

    pltpu.sync_copy(pos_hbm, pos_v)

    def compute_quarter(q, buf):
        def per_token(t, _):
            for k in range(DP // LANES):
                sl = pl.ds(k * LANES, LANES)
                buf[t, sl] = buf[t, sl] + pos_v[q * QL + t, sl]
            return 0

        lax.fori_loop(0, QL, per_token, 0)

    def do_row(r, _):
        row = base + r
        pltpu.sync_copy(x_hbm.at[row], idx_v)
        for i in range(L // LANES):
            sl = pl.ds(i * LANES, LANES)
            idx2_v[sl] = lax.shift_right_logical(idx_v[sl], 1)

        def gather(q, buf):
            return pltpu.async_copy(
                pair_hbm.at[idx2_v.at[pl.ds(q * QL, QL)]], buf, sem_g
            )

        def store(q, buf):
            return pltpu.async_copy(
                buf, out_hbm.at[row, pl.ds(q * QL, QL)], sem_w
            )

        g0 = gather(0, bufa)
        g1 = gather(1, bufb)
        g0.wait()
        compute_quarter(0, bufa)
        w0 = store(0, bufa)
        g1.wait()
        compute_quarter(1, bufb)
        w1 = store(1, bufb)
        w0.wait()
        g2 = gather(2, bufa)
        w1.wait()
        g3 = gather(3, bufb)
        g2.wait()
        compute_quarter(2, bufa)
        w2 = store(2, bufa)
        g3.wait()
        compute_quarter(3, bufb)
        w3 = store(3, bufb)
        w2.wait()
        w3.wait()
        return 0

    lax.fori_loop(0, ROWS_PER_W, do_row, 0)


_gather = functools.partial(
    pl.kernel,
    out_type=jax.ShapeDtypeStruct((B, L, DP), jnp.float32),
    mesh=plsc.VectorSubcoreMesh(core_axis_name="c", subcore_axis_name="s"),
    scratch_types=[
        pltpu.VMEM((L,), jnp.int32),
        pltpu.VMEM((L,), jnp.int32),
        pltpu.VMEM((QL, DP), jnp.float32),
        pltpu.VMEM((QL, DP), jnp.float32),
        pltpu.VMEM((L, DP), jnp.float32),
        pltpu.SemaphoreType.DMA,
        pltpu.SemaphoreType.DMA,
    ],
    compiler_params=pltpu.CompilerParams(
        use_tc_tiling_on_sc=True, needs_layout_passes=False
    ),
)(_gather_body)


@jax.jit
def kernel(x, token_emb, pos_emb):
    xi = x.astype(jnp.int32)
    pair = _relayout(token_emb.T)
    pos2 = jnp.concatenate([pos_emb, pos_emb], axis=1)
    out2 = _gather(xi, pair, pos2)
    odd = (xi & 1)[:, :, None] == 1
    return jnp.where(odd, out2[:, :, D:], out2[:, :, :D])


# speedup vs baseline: 2.4205x; 2.4205x over previous
import functools

import jax
import jax.numpy as jnp
from jax import lax
from jax.experimental import pallas as pl
from jax.experimental.pallas import tpu as pltpu
from jax.experimental.pallas import tpu_sc as plsc

B, L, D = 1024, 512, 64
DP = 128
NC, NS = 2, 16
NW = NC * NS
ROWS_PER_W = B // NW
NQ = 4
QL = L // NQ
LANES = 16


def _body(x_hbm, tok_hbm, pos_hbm, out_hbm,
          idx_v, bufa, bufb, pos_v, sem_g, sem_w):
    c = lax.axis_index("c")
    s = lax.axis_index("s")
    wid = s * NC + c
    base = wid * ROWS_PER_W

    pltpu.sync_copy(pos_hbm, pos_v)

    def compute_quarter(q, buf):
        def per_token(t, _):
            for k in range(D // LANES):
                sl = pl.ds(k * LANES, LANES)
                buf[t, sl] = buf[t, sl] + pos_v[q * QL + t, sl]
            return 0

        lax.fori_loop(0, QL, per_token, 0)

    def do_row(r, _):
        row = base + r
        pltpu.sync_copy(x_hbm.at[row], idx_v)

        def gather(q, buf):
            return pltpu.async_copy(
                tok_hbm.at[idx_v.at[pl.ds(q * QL, QL)]], buf, sem_g
            )

        def store(q, buf):
            return pltpu.async_copy(
                buf, out_hbm.at[row, pl.ds(q * QL, QL)], sem_w
            )

        g0 = gather(0, bufa)
        g1 = gather(1, bufb)
        g0.wait()
        compute_quarter(0, bufa)
        w0 = store(0, bufa)
        g1.wait()
        compute_quarter(1, bufb)
        w1 = store(1, bufb)
        w0.wait()
        g2 = gather(2, bufa)
        w1.wait()
        g3 = gather(3, bufb)
        g2.wait()
        compute_quarter(2, bufa)
        w2 = store(2, bufa)
        g3.wait()
        compute_quarter(3, bufb)
        w3 = store(3, bufb)
        w2.wait()
        w3.wait()
        return 0

    lax.fori_loop(0, ROWS_PER_W, do_row, 0)


_emb = functools.partial(
    pl.kernel,
    out_type=jax.ShapeDtypeStruct((B, L, DP), jnp.float32),
    mesh=plsc.VectorSubcoreMesh(core_axis_name="c", subcore_axis_name="s"),
    scratch_types=[
        pltpu.VMEM((L,), jnp.int32),
        pltpu.VMEM((QL, DP), jnp.float32),
        pltpu.VMEM((QL, DP), jnp.float32),
        pltpu.VMEM((L, D), jnp.float32),
        pltpu.SemaphoreType.DMA,
        pltpu.SemaphoreType.DMA,
    ],
    compiler_params=pltpu.CompilerParams(
        use_tc_tiling_on_sc=True, needs_layout_passes=False
    ),
)(_body)


@jax.jit
def kernel(x, token_emb, pos_emb):
    tok_pad = jnp.concatenate([token_emb, token_emb], axis=1)
    out = _emb(x.astype(jnp.int32), tok_pad, pos_emb)
    return out[:, :, :D]


# speedup vs baseline: 2.7380x; 1.1312x over previous
import functools

import jax
import jax.numpy as jnp
from jax import lax
from jax.experimental import pallas as pl
from jax.experimental.pallas import tpu as pltpu
from jax.experimental.pallas import tpu_sc as plsc

B, L, D = 1024, 512, 64
DP = 128
NC, NS = 2, 16
NW = NC * NS
ROWS_PER_W = B // NW
NQ = 4
QL = L // NQ
LANES = 16


def _body(x_hbm, tok_hbm, pos_hbm, out_hbm,
          idx_v, bufa, bufb, pos_v, sem_g, sem_w):
    c = lax.axis_index("c")
    s = lax.axis_index("s")
    wid = s * NC + c
    base = wid * ROWS_PER_W

    pltpu.sync_copy(pos_hbm, pos_v)

    def compute_quarter(q, buf):
        def per_token(t, _):
            for k in range(D // LANES):
                sl = pl.ds(k * LANES, LANES)
                buf[t, sl] = buf[t, sl] + pos_v[q * QL + t, sl]
            return 0

        lax.fori_loop(0, QL, per_token, 0)

    def do_row(r, _):
        row = base + r
        pltpu.sync_copy(x_hbm.at[row], idx_v)

        def gather(q, buf):
            return pltpu.async_copy(
                tok_hbm.at[idx_v.at[pl.ds(q * QL, QL)]], buf, sem_g
            )

        def store(q, buf):
            return pltpu.async_copy(
                buf, out_hbm.at[row, pl.ds(q * QL, QL)], sem_w
            )

        g0 = gather(0, bufa)
        g0.wait()
        g1 = gather(1, bufb)
        compute_quarter(0, bufa)
        w0 = store(0, bufa)
        g1.wait()
        w0.wait()
        g2 = gather(2, bufa)
        compute_quarter(1, bufb)
        w1 = store(1, bufb)
        g2.wait()
        w1.wait()
        g3 = gather(3, bufb)
        compute_quarter(2, bufa)
        w2 = store(2, bufa)
        g3.wait()
        compute_quarter(3, bufb)
        w3 = store(3, bufb)
        w2.wait()
        w3.wait()
        return 0

    lax.fori_loop(0, ROWS_PER_W, do_row, 0)


_emb = functools.partial(
    pl.kernel,
    out_type=jax.ShapeDtypeStruct((B, L, DP), jnp.float32),
    mesh=plsc.VectorSubcoreMesh(core_axis_name="c", subcore_axis_name="s"),
    scratch_types=[
        pltpu.VMEM((L,), jnp.int32),
        pltpu.VMEM((QL, DP), jnp.float32),
        pltpu.VMEM((QL, DP), jnp.float32),
        pltpu.VMEM((L, DP), jnp.float32),
        pltpu.SemaphoreType.DMA,
        pltpu.SemaphoreType.DMA,
    ],
    compiler_params=pltpu.CompilerParams(use_tc_tiling_on_sc=True),
)(_body)


@jax.jit
def kernel(x, token_emb, pos_emb):
    tok_pad = jnp.pad(token_emb, ((0, 0), (0, DP - D)))
    pos_pad = jnp.pad(pos_emb, ((0, 0), (0, DP - D)))
    out = _emb(x.astype(jnp.int32), tok_pad, pos_pad)
    return out[:, :, :D]


# speedup vs baseline: 2.7439x; 1.0021x over previous
import functools

import jax
import jax.numpy as jnp
from jax import lax
from jax.experimental import pallas as pl
from jax.experimental.pallas import tpu as pltpu
from jax.experimental.pallas import tpu_sc as plsc

B, L, D = 1024, 512, 64
DP = 128
NC, NS = 2, 16
NW = NC * NS
ROWS_PER_W = B // NW
NQ = 4
QL = L // NQ
LANES = 16


def _body(x_hbm, tok_hbm, pos_hbm, out_hbm,
          idx_v, bufa, bufb, pos_v, sem_g, sem_w):
    c = lax.axis_index("c")
    s = lax.axis_index("s")
    wid = s * NC + c
    base = wid * ROWS_PER_W

    pltpu.sync_copy(pos_hbm, pos_v)

    def compute_quarter(q, buf):
        def per_token(t):
            for k in range(D // LANES):
                sl = pl.ds(k * LANES, LANES)
                buf[t, sl] = buf[t, sl] + pos_v[q * QL + t, sl]

        plsc.parallel_loop(0, QL, 1, unroll=4)(per_token)

    def do_row(r, _):
        row = base + r
        pltpu.sync_copy(x_hbm.at[row], idx_v)

        def gather(q, buf):
            return pltpu.async_copy(
                tok_hbm.at[idx_v.at[pl.ds(q * QL, QL)]], buf, sem_g
            )

        def store(q, buf):
            return pltpu.async_copy(
                buf, out_hbm.at[row, pl.ds(q * QL, QL)], sem_w
            )

        g0 = gather(0, bufa)
        g0.wait()
        g1 = gather(1, bufb)
        compute_quarter(0, bufa)
        w0 = store(0, bufa)
        g1.wait()
        w0.wait()
        g2 = gather(2, bufa)
        compute_quarter(1, bufb)
        w1 = store(1, bufb)
        g2.wait()
        w1.wait()
        g3 = gather(3, bufb)
        compute_quarter(2, bufa)
        w2 = store(2, bufa)
        g3.wait()
        compute_quarter(3, bufb)
        w3 = store(3, bufb)
        w2.wait()
        w3.wait()
        return 0

    lax.fori_loop(0, ROWS_PER_W, do_row, 0)


_emb = functools.partial(
    pl.kernel,
    out_type=jax.ShapeDtypeStruct((B, L, DP), jnp.float32),
    mesh=plsc.VectorSubcoreMesh(core_axis_name="c", subcore_axis_name="s"),
    scratch_types=[
        pltpu.VMEM((L,), jnp.int32),
        pltpu.VMEM((QL, DP), jnp.float32),
        pltpu.VMEM((QL, DP), jnp.float32),
        pltpu.VMEM((L, DP), jnp.float32),
        pltpu.SemaphoreType.DMA,
        pltpu.SemaphoreType.DMA,
    ],
    compiler_params=pltpu.CompilerParams(use_tc_tiling_on_sc=True),
)(_body)


@jax.jit
def kernel(x, token_emb, pos_emb):
    tok_pad = jnp.pad(token_emb, ((0, 0), (0, DP - D)))
    pos_pad = jnp.pad(pos_emb, ((0, 0), (0, DP - D)))
    out = _emb(x.astype(jnp.int32), tok_pad, pos_pad)
    return out[:, :, :D]


# speedup vs baseline: 2.8338x; 1.0327x over previous
import functools

import jax
import jax.numpy as jnp
from jax import lax
from jax.experimental import pallas as pl
from jax.experimental.pallas import tpu as pltpu
from jax.experimental.pallas import tpu_sc as plsc

B, L, D = 1024, 512, 64
DP = 128
NC, NS = 2, 16
NW = NC * NS
ROWS_PER_W = B // NW
NQ = 4
QL = L // NQ
LANES = 16


def _body(x_hbm, tok_hbm, pos_hbm, out_hbm,
          idx_a, idx_b, bufa, bufb, pos_v, sem_g, sem_w):
    c = lax.axis_index("c")
    s = lax.axis_index("s")
    wid = s * NC + c
    base = wid * ROWS_PER_W

    pltpu.sync_copy(pos_hbm, pos_v)
    pltpu.sync_copy(x_hbm.at[base], idx_a)

    def compute_quarter(q, buf):
        def per_token(t):
            for k in range(D // LANES):
                sl = pl.ds(k * LANES, LANES)
                buf[t, sl] = buf[t, sl] + pos_v[q * QL + t, sl]

        plsc.parallel_loop(0, QL, 1, unroll=4)(per_token)

    def do_row_with(row, r, idx_v, idx_n):
        def gather(q, buf):
            return pltpu.async_copy(
                tok_hbm.at[idx_v.at[pl.ds(q * QL, QL)]], buf, sem_g
            )

        def store(q, buf):
            return pltpu.async_copy(
                buf, out_hbm.at[row, pl.ds(q * QL, QL)], sem_w
            )

        g0 = gather(0, bufa)
        g1 = gather(1, bufb)

        @pl.when(r + 1 < ROWS_PER_W)
        def _prefetch():
            pltpu.sync_copy(x_hbm.at[row + 1], idx_n)

        g0.wait()
        compute_quarter(0, bufa)
        w0 = store(0, bufa)
        g1.wait()
        w0.wait()
        g2 = gather(2, bufa)
        compute_quarter(1, bufb)
        w1 = store(1, bufb)
        g2.wait()
        w1.wait()
        g3 = gather(3, bufb)
        compute_quarter(2, bufa)
        w2 = store(2, bufa)
        g3.wait()
        compute_quarter(3, bufb)
        w3 = store(3, bufb)
        w2.wait()
        w3.wait()

    def do_row(r, _):
        row = base + r

        @pl.when(r % 2 == 0)
        def _even():
            do_row_with(row, r, idx_a, idx_b)

        @pl.when(r % 2 == 1)
        def _odd():
            do_row_with(row, r, idx_b, idx_a)

        return 0

    lax.fori_loop(0, ROWS_PER_W, do_row, 0)


_emb = functools.partial(
    pl.kernel,
    out_type=jax.ShapeDtypeStruct((B, L, DP), jnp.float32),
    mesh=plsc.VectorSubcoreMesh(core_axis_name="c", subcore_axis_name="s"),
    scratch_types=[
        pltpu.VMEM((L,), jnp.int32),
        pltpu.VMEM((L,), jnp.int32),
        pltpu.VMEM((QL, DP), jnp.float32),
        pltpu.VMEM((QL, DP), jnp.float32),
        pltpu.VMEM((L, DP), jnp.float32),
        pltpu.SemaphoreType.DMA,
        pltpu.SemaphoreType.DMA,
    ],
    compiler_params=pltpu.CompilerParams(use_tc_tiling_on_sc=True),
)(_body)


@jax.jit
def kernel(x, token_emb, pos_emb):
    tok_pad = jnp.pad(token_emb, ((0, 0), (0, DP - D)))
    pos_pad = jnp.pad(pos_emb, ((0, 0), (0, DP - D)))
    out = _emb(x.astype(jnp.int32), tok_pad, pos_pad)
    return out[:, :, :D]


# speedup vs baseline: 2.9489x; 1.0406x over previous
import functools

import jax
import jax.numpy as jnp
from jax import lax
from jax.experimental import pallas as pl
from jax.experimental.pallas import tpu as pltpu
from jax.experimental.pallas import tpu_sc as plsc

B, L, D = 1024, 512, 64
DP = 128
NC, NS = 2, 16
NW = NC * NS
ROWS_PER_W = B // NW
NQ = 4
QL = L // NQ
LANES = 16


def _body(x_hbm, tok_hbm, pos_hbm, out_hbm,
          idx_a, idx_b, bufa, bufb, bufc, pos_v, sem_g, sem_w):
    c = lax.axis_index("c")
    s = lax.axis_index("s")
    wid = s * NC + c
    base = wid * ROWS_PER_W

    pltpu.sync_copy(pos_hbm, pos_v)
    pltpu.sync_copy(x_hbm.at[base], idx_a)

    def compute_quarter(q, buf):
        def per_token(t):
            for k in range(D // LANES):
                sl = pl.ds(k * LANES, LANES)
                buf[t, sl] = buf[t, sl] + pos_v[q * QL + t, sl]

        plsc.parallel_loop(0, QL, 1, unroll=4)(per_token)

    def do_row_with(row, r, idx_v, idx_n):
        def gather(q, buf):
            return pltpu.async_copy(
                tok_hbm.at[idx_v.at[pl.ds(q * QL, QL)]], buf, sem_g
            )

        def store(q, buf):
            return pltpu.async_copy(
                buf, out_hbm.at[row, pl.ds(q * QL, QL)], sem_w
            )

        g0 = gather(0, bufa)
        g1 = gather(1, bufb)
        g2 = gather(2, bufc)

        @pl.when(r + 1 < ROWS_PER_W)
        def _prefetch():
            pltpu.sync_copy(x_hbm.at[row + 1], idx_n)

        g0.wait()
        compute_quarter(0, bufa)
        w0 = store(0, bufa)
        g1.wait()
        compute_quarter(1, bufb)
        w1 = store(1, bufb)
        w0.wait()
        g3 = gather(3, bufa)
        g2.wait()
        compute_quarter(2, bufc)
        w2 = store(2, bufc)
        g3.wait()
        compute_quarter(3, bufa)
        w3 = store(3, bufa)
        w1.wait()
        w2.wait()
        w3.wait()

    def do_row(r, _):
        row = base + r

        @pl.when(r % 2 == 0)
        def _even():
            do_row_with(row, r, idx_a, idx_b)

        @pl.when(r % 2 == 1)
        def _odd():
            do_row_with(row, r, idx_b, idx_a)

        return 0

    lax.fori_loop(0, ROWS_PER_W, do_row, 0)


_emb = functools.partial(
    pl.kernel,
    out_type=jax.ShapeDtypeStruct((B, L, DP), jnp.float32),
    mesh=plsc.VectorSubcoreMesh(core_axis_name="c", subcore_axis_name="s"),
    scratch_types=[
        pltpu.VMEM((L,), jnp.int32),
        pltpu.VMEM((L,), jnp.int32),
        pltpu.VMEM((QL, DP), jnp.float32),
        pltpu.VMEM((QL, DP), jnp.float32),
        pltpu.VMEM((QL, DP), jnp.float32),
        pltpu.VMEM((L, DP), jnp.float32),
        pltpu.SemaphoreType.DMA,
        pltpu.SemaphoreType.DMA,
    ],
    compiler_params=pltpu.CompilerParams(use_tc_tiling_on_sc=True),
)(_body)


@jax.jit
def kernel(x, token_emb, pos_emb):
    tok_pad = jnp.pad(token_emb, ((0, 0), (0, DP - D)))
    pos_pad = jnp.pad(pos_emb, ((0, 0), (0, DP - D)))
    out = _emb(x.astype(jnp.int32), tok_pad, pos_pad)
    return out[:, :, :D]
